# baseline (speedup 1.0000x reference)
import jax
import jax.numpy as jnp
from jax import lax
from jax.experimental import pallas as pl
from jax.experimental.pallas import tpu as pltpu
from jax.experimental.pallas import tpu_sc as plsc

_B = 16384
_E = 32
_NC = 2
_NS = 16
_NW = _NC * _NS
_PW = _B // _NW
_CH = 128
_NCH = _PW // _CH
_L = 16
_NG = _PW // _L


def _body(user_h, item_h, uemb_h, iemb_h, ubias_h, ibias_h, out_h,
          uidx_v, iidx_v, urows_v, irows_v, ub_v, ib_v, res_v, sem):
    wid = lax.axis_index("s") * _NC + lax.axis_index("c")
    base = wid * _PW

    pltpu.sync_copy(user_h.at[pl.ds(base, _PW)], uidx_v)
    pltpu.sync_copy(item_h.at[pl.ds(base, _PW)], iidx_v)

    copies = []
    for c in range(_NCH):
        sl = pl.ds(c * _CH, _CH)
        copies.append(pltpu.async_copy(uemb_h.at[uidx_v.at[sl]], urows_v.at[sl], sem))
        copies.append(pltpu.async_copy(iemb_h.at[iidx_v.at[sl]], irows_v.at[sl], sem))
        copies.append(pltpu.async_copy(ubias_h.at[uidx_v.at[sl]], ub_v.at[sl], sem))
        copies.append(pltpu.async_copy(ibias_h.at[iidx_v.at[sl]], ib_v.at[sl], sem))
    for cp in copies:
        cp.wait()

    lane = lax.iota(jnp.int32, _L)

    def group(g, carry):
        off = g * _L
        rows = off + lane
        acc = ub_v[pl.ds(off, _L)] + ib_v[pl.ds(off, _L)]
        for k in range(_E):
            col = jnp.full((_L,), k, jnp.int32)
            acc = acc + (plsc.load_gather(urows_v, [rows, col])
                         * plsc.load_gather(irows_v, [rows, col]))
        res_v[pl.ds(off, _L)] = acc
        return carry

    lax.fori_loop(0, _NG, group, 0)

    pltpu.sync_copy(res_v, out_h.at[pl.ds(base, _PW)])


@jax.jit
def _sc_score(user, item, uemb, iemb, ubias, ibias):
    mesh = plsc.VectorSubcoreMesh(core_axis_name="c", subcore_axis_name="s")
    return pl.kernel(
        _body,
        out_type=jax.ShapeDtypeStruct((_B,), jnp.float32),
        mesh=mesh,
        compiler_params=pltpu.CompilerParams(needs_layout_passes=False,
                                             use_tc_tiling_on_sc=False),
        scratch_types=[
            pltpu.VMEM((_PW,), jnp.int32),
            pltpu.VMEM((_PW,), jnp.int32),
            pltpu.VMEM((_PW, _E), jnp.float32),
            pltpu.VMEM((_PW, _E), jnp.float32),
            pltpu.VMEM((_PW,), jnp.float32),
            pltpu.VMEM((_PW,), jnp.float32),
            pltpu.VMEM((_PW,), jnp.float32),
            pltpu.SemaphoreType.DMA,
        ],
    )(user, item, uemb, iemb, ubias, ibias)


def kernel(user, item, u_embed, i_embed, u_bias, i_bias):
    return _sc_score(user.astype(jnp.int32), item.astype(jnp.int32),
                     u_embed, i_embed,
                     u_bias.reshape(-1), i_bias.reshape(-1))


# speedup vs baseline: 1.0008x; 1.0008x over previous
import jax
import jax.numpy as jnp
from jax import lax
from jax.experimental import pallas as pl
from jax.experimental.pallas import tpu as pltpu
from jax.experimental.pallas import tpu_sc as plsc

_B = 16384
_E = 32
_NC = 2
_NS = 16
_NW = _NC * _NS
_PW = _B // _NW
_CH = 128
_NCH = _PW // _CH
_L = 16
_NG = _PW // _L


def _body(user_h, item_h, uemb_h, iemb_h, ubias_h, ibias_h, out_h,
          uidx_v, iidx_v, urows_v, irows_v, ub_v, ib_v, res_v, sem):
    wid = lax.axis_index("s") * _NC + lax.axis_index("c")
    base = wid * _PW

    pltpu.sync_copy(user_h.at[pl.ds(base, _PW)], uidx_v)
    pltpu.sync_copy(item_h.at[pl.ds(base, _PW)], iidx_v)

    copies = []
    for c in range(_NCH):
        sl = pl.ds(c * _CH, _CH)
        copies.append(pltpu.async_copy(uemb_h.at[uidx_v.at[sl]], urows_v.at[sl], sem))
        copies.append(pltpu.async_copy(iemb_h.at[iidx_v.at[sl]], irows_v.at[sl], sem))
        copies.append(pltpu.async_copy(ubias_h.at[uidx_v.at[sl]], ub_v.at[sl], sem))
        copies.append(pltpu.async_copy(ibias_h.at[iidx_v.at[sl]], ib_v.at[sl], sem))
    for cp in copies:
        cp.wait()

    lane = lax.iota(jnp.int32, _L)

    def group(g, carry):
        p0 = g * _L
        rows = p0 + lane
        acc = ub_v[pl.ds(p0, _L)] + ib_v[pl.ds(p0, _L)]
        for k in range(_E):
            col = jnp.full((_L,), k, jnp.int32)
            acc = acc + (plsc.load_gather(urows_v, [rows, col])
                         * plsc.load_gather(irows_v, [rows, col]))
        res_v[pl.ds(p0, _L)] = acc
        return carry

    lax.fori_loop(0, _NG, group, 0)

    pltpu.sync_copy(res_v, out_h.at[pl.ds(base, _PW)])


@jax.jit
def _sc_score(user, item, uemb, iemb, ubias, ibias):
    mesh = plsc.VectorSubcoreMesh(core_axis_name="c", subcore_axis_name="s")
    return pl.kernel(
        _body,
        out_type=jax.ShapeDtypeStruct((_B,), jnp.float32),
        mesh=mesh,
        compiler_params=pltpu.CompilerParams(needs_layout_passes=False,
                                             use_tc_tiling_on_sc=False),
        scratch_types=[
            pltpu.VMEM((_PW,), jnp.int32),
            pltpu.VMEM((_PW,), jnp.int32),
            pltpu.VMEM((_PW, _E), jnp.float32),
            pltpu.VMEM((_PW, _E), jnp.float32),
            pltpu.VMEM((_PW,), jnp.float32),
            pltpu.VMEM((_PW,), jnp.float32),
            pltpu.VMEM((_PW,), jnp.float32),
            pltpu.SemaphoreType.DMA,
        ],
    )(user, item, uemb, iemb, ubias, ibias)


def kernel(user, item, u_embed, i_embed, u_bias, i_bias):
    return _sc_score(user.astype(jnp.int32), item.astype(jnp.int32),
                     u_embed, i_embed,
                     u_bias.reshape(-1), i_bias.reshape(-1))


# speedup vs baseline: 2.3064x; 2.3046x over previous
import jax
import jax.numpy as jnp
from jax import lax
from jax.experimental import pallas as pl
from jax.experimental.pallas import tpu as pltpu
from jax.experimental.pallas import tpu_sc as plsc

_B = 16384
_E = 32
_N = 1000000
_NC = 2
_NS = 16
_NW = _NC * _NS
_PW = _B // _NW
_CH = 128
_NCH = _PW // _CH
_L = 16
_NG = _PW // _L
_NB = _E // 8


def _fire(tab_h, tile_v, buf, r, sem):
    tc = pl.multiple_of((r >> 7) * 128, 128)
    for b in range(_NB):
        pltpu.async_copy(tab_h.at[pl.ds(b * 8, 8), pl.ds(tc, 128)],
                         tile_v.at[buf, b], sem)


def _drain(tab_h, tile_v, buf, sem):
    for b in range(_NB):
        pltpu.make_async_copy(tab_h.at[pl.ds(0, 8), pl.ds(0, 128)],
                              tile_v.at[buf, b], sem).wait()


def _body(user_h, item_h, uembT_h, iembT_h, ubias_h, ibias_h, out_h,
          uidx_v, iidx_v, utile_v, itile_v, ub_v, ib_v, part_v, res_v, sem,
          bsem):
    wid = lax.axis_index("s") * _NC + lax.axis_index("c")
    base = wid * _PW

    pltpu.sync_copy(user_h.at[pl.ds(base, _PW)], uidx_v)
    pltpu.sync_copy(item_h.at[pl.ds(base, _PW)], iidx_v)

    bias_copies = []
    for c in range(_NCH):
        sl = pl.ds(c * _CH, _CH)
        bias_copies.append(pltpu.async_copy(ubias_h.at[uidx_v.at[sl]], ub_v.at[sl], bsem))
        bias_copies.append(pltpu.async_copy(ibias_h.at[iidx_v.at[sl]], ib_v.at[sl], bsem))

    lane = lax.iota(jnp.int32, _L)
    half_idx = []
    for h in range(2):
        cs = h * _L + lane
        half_idx.append(((cs >> 3), (cs & 7)))

    v0u = uidx_v[pl.ds(0, _L)]
    v0i = iidx_v[pl.ds(0, _L)]
    _fire(uembT_h, utile_v, 0, v0u[0], sem)
    _fire(iembT_h, itile_v, 0, v0i[0], sem)

    def group(g, carry):
        p0 = g * _L
        ru = uidx_v[pl.ds(p0, _L)]
        ri = iidx_v[pl.ds(p0, _L)]
        pn0 = jnp.minimum(p0 + _L, _PW - _L)
        run = uidx_v[pl.ds(pn0, _L)]
        rin = iidx_v[pl.ds(pn0, _L)]
        for j in range(_L):
            p = p0 + j
            buf = j & 1
            nbuf = (j + 1) & 1
            if True:
                rnu = run[0] if j == _L - 1 else ru[j + 1]
                rni = rin[0] if j == _L - 1 else ri[j + 1]
                pn = p + 1

                @pl.when(pn < _PW)
                def _():
                    _fire(uembT_h, utile_v, nbuf, rnu, sem)
                    _fire(iembT_h, itile_v, nbuf, rni, sem)

            _drain(uembT_h, utile_v, buf, sem)
            _drain(iembT_h, itile_v, buf, sem)
            rlu = ru[j] & 127
            rli = ri[j] & 127
            acc = jnp.zeros((_L,), jnp.float32)
            for h in range(2):
                bv, cv = half_idx[h]
                uv = plsc.load_gather(
                    utile_v, [jnp.full((_L,), buf, jnp.int32), bv, cv,
                              jnp.full((_L,), rlu, jnp.int32)])
                iv = plsc.load_gather(
                    itile_v, [jnp.full((_L,), buf, jnp.int32), bv, cv,
                              jnp.full((_L,), rli, jnp.int32)])
                acc = acc + uv * iv
            part_v[p, pl.ds(0, _L)] = acc
        return carry

    lax.fori_loop(0, _NG, group, 0, unroll=False)

    for cp in bias_copies:
        cp.wait()

    zeros = jnp.zeros((_L,), jnp.int32)

    def bgrp(g, carry):
        p0 = g * _L
        rows = p0 + lane
        acc = ub_v[pl.ds(p0, _L)] + ib_v[pl.ds(p0, _L)]
        for k in range(_L):
            ks = jnp.full((_L,), k, jnp.int32)
            acc = acc + plsc.load_gather(part_v, [rows, ks])
        res_v[pl.ds(p0, _L)] = acc
        return carry

    lax.fori_loop(0, _NG, bgrp, 0)

    pltpu.sync_copy(res_v, out_h.at[pl.ds(base, _PW)])


@jax.jit
def _sc_score(user, item, uembT, iembT, ubias, ibias):
    mesh = plsc.VectorSubcoreMesh(core_axis_name="c", subcore_axis_name="s")
    return pl.kernel(
        _body,
        out_type=jax.ShapeDtypeStruct((_B,), jnp.float32),
        mesh=mesh,
        compiler_params=pltpu.CompilerParams(needs_layout_passes=False,
                                             use_tc_tiling_on_sc=True),
        scratch_types=[
            pltpu.VMEM((_PW,), jnp.int32),
            pltpu.VMEM((_PW,), jnp.int32),
            pltpu.VMEM((2, _NB, 8, 128), jnp.float32),
            pltpu.VMEM((2, _NB, 8, 128), jnp.float32),
            pltpu.VMEM((_PW,), jnp.float32),
            pltpu.VMEM((_PW,), jnp.float32),
            pltpu.VMEM((_PW, _L), jnp.float32),
            pltpu.VMEM((_PW,), jnp.float32),
            pltpu.SemaphoreType.DMA,
            pltpu.SemaphoreType.DMA,
        ],
    )(user, item, uembT, iembT, ubias, ibias)


def kernel(user, item, u_embed, i_embed, u_bias, i_bias):
    return _sc_score(user.astype(jnp.int32), item.astype(jnp.int32),
                     u_embed.T, i_embed.T,
                     u_bias.reshape(-1), i_bias.reshape(-1))


# speedup vs baseline: 2.8261x; 1.2253x over previous
import jax
import jax.numpy as jnp
from jax import lax
from jax.experimental import pallas as pl
from jax.experimental.pallas import tpu as pltpu
from jax.experimental.pallas import tpu_sc as plsc

_B = 16384
_E = 32
_NC = 2
_NS = 16
_NW = _NC * _NS
_PW = _B // _NW
_CH = 128
_NCH = _PW // _CH
_L = 16
_NG = _PW // _L


def _fire(tab_h, tile_v, buf, r, sem):
    tc = pl.multiple_of((r >> 7) * 128, 128)
    pltpu.async_copy(tab_h.at[pl.ds(0, _E), pl.ds(tc, 128)],
                     tile_v.at[buf], sem)


def _drain(tab_h, tile_v, buf, sem):
    pltpu.make_async_copy(tab_h.at[pl.ds(0, _E), pl.ds(0, 128)],
                          tile_v.at[buf], sem).wait()


def _body(user_h, item_h, uembT_h, iembT_h, ubias_h, ibias_h, out_h,
          uidx_v, iidx_v, utile_v, itile_v, ub_v, ib_v, part_v, res_v, sem,
          bsem):
    wid = lax.axis_index("s") * _NC + lax.axis_index("c")
    base = wid * _PW

    pltpu.sync_copy(user_h.at[pl.ds(base, _PW)], uidx_v)
    pltpu.sync_copy(item_h.at[pl.ds(base, _PW)], iidx_v)

    bias_copies = []
    for c in range(_NCH):
        sl = pl.ds(c * _CH, _CH)
        bias_copies.append(pltpu.async_copy(ubias_h.at[uidx_v.at[sl]], ub_v.at[sl], bsem))
        bias_copies.append(pltpu.async_copy(ibias_h.at[iidx_v.at[sl]], ib_v.at[sl], bsem))

    lane = lax.iota(jnp.int32, _L)
    half_cs = [h * _L + lane for h in range(2)]

    v0u = uidx_v[pl.ds(0, _L)]
    v0i = iidx_v[pl.ds(0, _L)]
    for pp in range(2):
        _fire(uembT_h, utile_v, pp, v0u[pp], sem)
        _fire(iembT_h, itile_v, pp, v0i[pp], sem)

    def group(g, carry):
        p0 = g * _L
        ru = uidx_v[pl.ds(p0, _L)]
        ri = iidx_v[pl.ds(p0, _L)]
        pn0 = jnp.minimum(p0 + _L, _PW - _L)
        run = uidx_v[pl.ds(pn0, _L)]
        rin = iidx_v[pl.ds(pn0, _L)]
        for j in range(_L):
            p = p0 + j
            buf = j & 3
            nbuf = (j + 2) & 3
            rnu = ru[j + 2] if j < _L - 2 else run[j - (_L - 2)]
            rni = ri[j + 2] if j < _L - 2 else rin[j - (_L - 2)]
            pn = p + 2

            @pl.when(pn < _PW)
            def _():
                _fire(uembT_h, utile_v, nbuf, rnu, sem)
                _fire(iembT_h, itile_v, nbuf, rni, sem)

            _drain(uembT_h, utile_v, buf, sem)
            _drain(iembT_h, itile_v, buf, sem)
            rlu = ru[j] & 127
            rli = ri[j] & 127
            acc = jnp.zeros((_L,), jnp.float32)
            for h in range(2):
                cs = half_cs[h]
                uv = plsc.load_gather(
                    utile_v, [jnp.full((_L,), buf, jnp.int32), cs,
                              jnp.full((_L,), rlu, jnp.int32)])
                iv = plsc.load_gather(
                    itile_v, [jnp.full((_L,), buf, jnp.int32), cs,
                              jnp.full((_L,), rli, jnp.int32)])
                acc = acc + uv * iv
            part_v[p, pl.ds(0, _L)] = acc
        return carry

    lax.fori_loop(0, _NG, group, 0)

    for cp in bias_copies:
        cp.wait()

    def bgrp(g, carry):
        p0 = g * _L
        rows = p0 + lane
        acc = ub_v[pl.ds(p0, _L)] + ib_v[pl.ds(p0, _L)]
        for k in range(_L):
            ks = jnp.full((_L,), k, jnp.int32)
            acc = acc + plsc.load_gather(part_v, [rows, ks])
        res_v[pl.ds(p0, _L)] = acc
        return carry

    lax.fori_loop(0, _NG, bgrp, 0)

    pltpu.sync_copy(res_v, out_h.at[pl.ds(base, _PW)])


@jax.jit
def _sc_score(user, item, uembT, iembT, ubias, ibias):
    mesh = plsc.VectorSubcoreMesh(core_axis_name="c", subcore_axis_name="s")
    return pl.kernel(
        _body,
        out_type=jax.ShapeDtypeStruct((_B,), jnp.float32),
        mesh=mesh,
        compiler_params=pltpu.CompilerParams(needs_layout_passes=False,
                                             use_tc_tiling_on_sc=True),
        scratch_types=[
            pltpu.VMEM((_PW,), jnp.int32),
            pltpu.VMEM((_PW,), jnp.int32),
            pltpu.VMEM((4, _E, 128), jnp.float32),
            pltpu.VMEM((4, _E, 128), jnp.float32),
            pltpu.VMEM((_PW,), jnp.float32),
            pltpu.VMEM((_PW,), jnp.float32),
            pltpu.VMEM((_PW, _L), jnp.float32),
            pltpu.VMEM((_PW,), jnp.float32),
            pltpu.SemaphoreType.DMA,
            pltpu.SemaphoreType.DMA,
        ],
    )(user, item, uembT, iembT, ubias, ibias)


def kernel(user, item, u_embed, i_embed, u_bias, i_bias):
    return _sc_score(user.astype(jnp.int32), item.astype(jnp.int32),
                     u_embed.T, i_embed.T,
                     u_bias.reshape(-1), i_bias.reshape(-1))


# speedup vs baseline: 3.0922x; 1.0942x over previous
import jax
import jax.numpy as jnp
from jax import lax
from jax.experimental import pallas as pl
from jax.experimental.pallas import tpu as pltpu
from jax.experimental.pallas import tpu_sc as plsc

_B = 16384
_E = 32
_NC = 2
_NS = 16
_NW = _NC * _NS
_PW = _B // _NW
_CH = 128
_NCH = _PW // _CH
_L = 16
_NG = _PW // _L


def _fire(tab_h, tile_v, buf, r, sem):
    tc = pl.multiple_of((r >> 7) * 128, 128)
    pltpu.async_copy(tab_h.at[pl.ds(0, _E), pl.ds(tc, 128)],
                     tile_v.at[buf], sem)


def _drain(tab_h, tile_v, buf, sem):
    pltpu.make_async_copy(tab_h.at[pl.ds(0, _E), pl.ds(0, 128)],
                          tile_v.at[buf], sem).wait()


def _body(user_h, item_h, uembT_h, iembT_h, ubias_h, ibias_h, out_h,
          uidx_v, iidx_v, utile_v, itile_v, ub_v, ib_v, part_v, res_v, sem,
          bsem):
    wid = lax.axis_index("s") * _NC + lax.axis_index("c")
    base = wid * _PW

    pltpu.sync_copy(user_h.at[pl.ds(base, _PW)], uidx_v)
    pltpu.sync_copy(item_h.at[pl.ds(base, _PW)], iidx_v)

    bias_copies = []
    for c in range(_NCH):
        sl = pl.ds(c * _CH, _CH)
        bias_copies.append(pltpu.async_copy(ubias_h.at[uidx_v.at[sl]], ub_v.at[sl], bsem))
        bias_copies.append(pltpu.async_copy(ibias_h.at[iidx_v.at[sl]], ib_v.at[sl], bsem))

    lane = lax.iota(jnp.int32, _L)
    half_cs = [h * _L + lane for h in range(2)]

    v0u = uidx_v[pl.ds(0, _L)]
    v0i = iidx_v[pl.ds(0, _L)]
    for pp in range(3):
        _fire(uembT_h, utile_v, pp, v0u[pp], sem)
        _fire(iembT_h, itile_v, pp, v0i[pp], sem)

    def group(g, carry):
        p0 = g * _L
        ru = uidx_v[pl.ds(p0, _L)]
        ri = iidx_v[pl.ds(p0, _L)]
        pn0 = jnp.minimum(p0 + _L, _PW - _L)
        run = uidx_v[pl.ds(pn0, _L)]
        rin = iidx_v[pl.ds(pn0, _L)]
        for j in range(_L):
            p = p0 + j
            buf = j & 3
            nbuf = (j + 3) & 3
            rnu = ru[j + 3] if j < _L - 3 else run[j - (_L - 3)]
            rni = ri[j + 3] if j < _L - 3 else rin[j - (_L - 3)]
            pn = p + 3

            @pl.when(pn < _PW)
            def _():
                _fire(uembT_h, utile_v, nbuf, rnu, sem)
                _fire(iembT_h, itile_v, nbuf, rni, sem)

            _drain(uembT_h, utile_v, buf, sem)
            _drain(iembT_h, itile_v, buf, sem)
            rlu = ru[j] & 127
            rli = ri[j] & 127
            acc = jnp.zeros((_L,), jnp.float32)
            for h in range(2):
                cs = half_cs[h]
                uv = plsc.load_gather(
                    utile_v, [jnp.full((_L,), buf, jnp.int32), cs,
                              jnp.full((_L,), rlu, jnp.int32)])
                iv = plsc.load_gather(
                    itile_v, [jnp.full((_L,), buf, jnp.int32), cs,
                              jnp.full((_L,), rli, jnp.int32)])
                acc = acc + uv * iv
            part_v[p, pl.ds(0, _L)] = acc
        return carry

    lax.fori_loop(0, _NG, group, 0)

    for cp in bias_copies:
        cp.wait()

    def bgrp(g, carry):
        p0 = g * _L
        rows = p0 + lane
        acc = ub_v[pl.ds(p0, _L)] + ib_v[pl.ds(p0, _L)]
        for k in range(_L):
            ks = jnp.full((_L,), k, jnp.int32)
            acc = acc + plsc.load_gather(part_v, [rows, ks])
        res_v[pl.ds(p0, _L)] = acc
        return carry

    lax.fori_loop(0, _NG, bgrp, 0)

    pltpu.sync_copy(res_v, out_h.at[pl.ds(base, _PW)])


@jax.jit
def _sc_score(user, item, uembT, iembT, ubias, ibias):
    mesh = plsc.VectorSubcoreMesh(core_axis_name="c", subcore_axis_name="s")
    return pl.kernel(
        _body,
        out_type=jax.ShapeDtypeStruct((_B,), jnp.float32),
        mesh=mesh,
        compiler_params=pltpu.CompilerParams(needs_layout_passes=False,
                                             use_tc_tiling_on_sc=True),
        scratch_types=[
            pltpu.VMEM((_PW,), jnp.int32),
            pltpu.VMEM((_PW,), jnp.int32),
            pltpu.VMEM((4, _E, 128), jnp.float32),
            pltpu.VMEM((4, _E, 128), jnp.float32),
            pltpu.VMEM((_PW,), jnp.float32),
            pltpu.VMEM((_PW,), jnp.float32),
            pltpu.VMEM((_PW, _L), jnp.float32),
            pltpu.VMEM((_PW,), jnp.float32),
            pltpu.SemaphoreType.DMA,
            pltpu.SemaphoreType.DMA,
        ],
    )(user, item, uembT, iembT, ubias, ibias)


def kernel(user, item, u_embed, i_embed, u_bias, i_bias):
    return _sc_score(user.astype(jnp.int32), item.astype(jnp.int32),
                     u_embed.T, i_embed.T,
                     u_bias.reshape(-1), i_bias.reshape(-1))
